# Initial kernel scaffold; baseline (speedup 1.0000x reference)
import jax
import jax.numpy as jnp
from jax.experimental import pallas as pl


def kernel(x, coeff):
    raise NotImplementedError("write your pallas kernel here")


# speedup vs baseline: 1592.1277x; 1592.1277x over previous
import jax
import jax.numpy as jnp
from jax import lax
from jax.experimental import pallas as pl
from jax.experimental.pallas import tpu as pltpu

_C = 96
_H = 224
_W = 224
_BINS = 64
_N = 4
_ROWS = _N * _C
_HW = _H * _W
_SUB = 392
_LANES = 128
_TOT = _ROWS * _HW


def _minmax_body(x_ref, mn_ref, mx_ref):
    i = pl.program_id(0)
    bm = jnp.min(x_ref[...])
    bM = jnp.max(x_ref[...])

    @pl.when(i == 0)
    def _init():
        mn_ref[0, 0] = bm
        mx_ref[0, 0] = bM

    @pl.when(i > 0)
    def _acc():
        mn_ref[0, 0] = jnp.minimum(mn_ref[0, 0], bm)
        mx_ref[0, 0] = jnp.maximum(mx_ref[0, 0], bM)


def _minmax(x_r, interpret=False):
    grid = 48
    blk = _ROWS // grid
    return pl.pallas_call(
        _minmax_body,
        grid=(grid,),
        in_specs=[pl.BlockSpec((blk, _SUB, _LANES), lambda i: (i, 0, 0))],
        out_specs=[
            pl.BlockSpec(memory_space=pltpu.SMEM),
            pl.BlockSpec(memory_space=pltpu.SMEM),
        ],
        out_shape=[
            jax.ShapeDtypeStruct((1, 1), jnp.float32),
            jax.ShapeDtypeStruct((1, 1), jnp.float32),
        ],
        compiler_params=pltpu.CompilerParams(
            dimension_semantics=("arbitrary",)),
        interpret=interpret,
    )(x_r)


def _scan_body(x_ref, tau_ref, p_ref, c_ref, carry_ref, cnt_ref):
    i = pl.program_id(0)

    @pl.when(i == 0)
    def _init():
        carry_ref[0] = 0.0
        cnt_ref[...] = jnp.zeros((1, _BINS), jnp.int32)

    blk = x_ref[0]

    t = jnp.tanh(blk)
    iu = lax.broadcasted_iota(jnp.int32, (_LANES, _LANES), 0)
    ju = lax.broadcasted_iota(jnp.int32, (_LANES, _LANES), 1)
    U = jnp.where(iu <= ju, 1.0, 0.0)
    cs = jnp.dot(t, U, preferred_element_type=jnp.float32,
                 precision=lax.Precision.HIGHEST)
    rt = cs[:, _LANES - 1:_LANES]
    il = lax.broadcasted_iota(jnp.int32, (_SUB, _SUB), 0)
    jl = lax.broadcasted_iota(jnp.int32, (_SUB, _SUB), 1)
    Ls = jnp.where(jl < il, 1.0, 0.0)
    ro = jnp.dot(Ls, rt, preferred_element_type=jnp.float32,
                 precision=lax.Precision.HIGHEST)
    carry = carry_ref[0]
    p_ref[0] = cs + ro + carry
    carry_ref[0] = carry + jnp.sum(t)

    ge = [jnp.float32(_HW)]
    for b in range(1, _BINS):
        ge.append(jnp.sum(jnp.where(blk >= tau_ref[b], 1.0, 0.0)))
    g = jnp.stack(ge)
    gnext = jnp.concatenate([g[1:], jnp.zeros((1,), jnp.float32)])
    hist = (g - gnext).astype(jnp.int32)
    new_cnt = cnt_ref[...] + hist.reshape(1, _BINS)
    cnt_ref[...] = new_cnt
    c_ref[0] = new_cnt


def _scan_counts(x_r, tau, interpret=False):
    return pl.pallas_call(
        _scan_body,
        grid=(_ROWS,),
        in_specs=[
            pl.BlockSpec((1, _SUB, _LANES), lambda i: (i, 0, 0)),
            pl.BlockSpec(memory_space=pltpu.SMEM),
        ],
        out_specs=[
            pl.BlockSpec((1, _SUB, _LANES), lambda i: (i, 0, 0)),
            pl.BlockSpec((1, 1, _BINS), lambda i: (i, 0, 0)),
        ],
        out_shape=[
            jax.ShapeDtypeStruct((_ROWS, _SUB, _LANES), jnp.float32),
            jax.ShapeDtypeStruct((_ROWS, 1, _BINS), jnp.int32),
        ],
        scratch_shapes=[
            pltpu.SMEM((1,), jnp.float32),
            pltpu.VMEM((1, _BINS), jnp.int32),
        ],
        compiler_params=pltpu.CompilerParams(
            dimension_semantics=("arbitrary",)),
        interpret=interpret,
    )(x_r, tau)


def _run(x, coeff, interpret=False):
    x_r = x.reshape(_ROWS, _SUB, _LANES)
    mn, mx = _minmax(x_r, interpret)
    tau = jnp.linspace(mn[0, 0], mx[0, 0], _BINS + 1)
    P, Ccum = _scan_counts(x_r, tau, interpret)
    Q = jnp.concatenate(
        [jnp.zeros((1, _BINS), jnp.int32), Ccum.reshape(_ROWS, _BINS)], axis=0)
    Pf = P.reshape(-1)
    g = jnp.where(Q > 0, jnp.take(Pf, jnp.maximum(Q - 1, 0)), 0.0)
    T = g[1:] - g[:-1]
    z = jnp.sum(T.reshape(_N, _C, _BINS) * coeff[None], axis=2)
    return z


def kernel(x, coeff):
    return _run(x, coeff, interpret=False)
